# Initial kernel scaffold; baseline (speedup 1.0000x reference)
import jax
import jax.numpy as jnp
from jax.experimental import pallas as pl


def kernel(src, time, weekday, emb_loc, minute_embed, hour_embed, weekday_embed):
    raise NotImplementedError("write your pallas kernel here")


# speedup vs baseline: 5.5865x; 5.5865x over previous
import math

import jax
import jax.numpy as jnp
from jax import lax
from jax.experimental import pallas as pl
from jax.experimental.pallas import tpu as pltpu
from jax.experimental.pallas import tpu_sc as plsc

SEQ_LEN = 200
BATCH = 1024
D = 64
MINUTE_SIZE = 4
HOUR_SIZE = 24
WEEKDAY_SIZE = 7

NUM_CORES = 2
NUM_SUBCORES = 16
NUM_WORKERS = NUM_CORES * NUM_SUBCORES

TOTAL = SEQ_LEN * BATCH
CHUNK = 128
NUM_CHUNKS = TOTAL // CHUNK
CHUNKS_PER_WORKER = NUM_CHUNKS // NUM_WORKERS
CHUNKS_PER_ROW = BATCH // CHUNK


def _pe_const(seq_len, emb_size):
    den = jnp.exp(-jnp.arange(0, emb_size, 2).astype(jnp.float32)
                  * math.log(10000.0) / emb_size)
    pos = jnp.arange(0, seq_len).reshape(seq_len, 1).astype(jnp.float32)
    pe = jnp.zeros((seq_len, emb_size), dtype=jnp.float32)
    pe = pe.at[:, 0::2].set(jnp.sin(pos * den))
    pe = pe.at[:, 1::2].set(jnp.cos(pos * den))
    return pe


def _sc_kernel(src_hbm, time_hbm, wk_hbm, emb_hbm, tw_hbm, pe_hbm, out_hbm,
               src_v, time_v, wk_v, twidx_v, loc_v, twr_v, pe_v,
               sem_a, sem_b):
    wid = lax.axis_index("s") * NUM_CORES + lax.axis_index("c")

    def chunk_body(c, _):
        q = wid * CHUNKS_PER_WORKER + c
        s = q // CHUNKS_PER_ROW
        base = q * CHUNK

        pltpu.sync_copy(src_hbm.at[pl.ds(base, CHUNK)], src_v)
        pltpu.sync_copy(time_hbm.at[pl.ds(base, CHUNK)], time_v)
        pltpu.sync_copy(wk_hbm.at[pl.ds(base, CHUNK)], wk_v)
        pltpu.sync_copy(pe_hbm.at[pl.ds(s * D, D)], pe_v)

        for i in range(CHUNK // 16):
            sl = pl.ds(i * 16, 16)
            twidx_v[sl] = time_v[sl] * WEEKDAY_SIZE + wk_v[sl]

        cp_a = pltpu.async_copy(emb_hbm.at[src_v], loc_v, sem_a)
        cp_b = pltpu.async_copy(tw_hbm.at[twidx_v], twr_v, sem_b)
        cp_a.wait()
        cp_b.wait()

        pe_regs = [pe_v[pl.ds(j * 16, 16)] for j in range(D // 16)]

        def row_body(r, _):
            for j in range(D // 16):
                sl = pl.ds(j * 16, 16)
                loc_v[r, sl] = (loc_v[r, sl] * 8.0 + twr_v[r, sl]
                                + pe_regs[j])
            return 0

        lax.fori_loop(0, CHUNK, row_body, 0, unroll=4)

        pltpu.sync_copy(loc_v, out_hbm.at[pl.ds(base, CHUNK)])
        return 0

    lax.fori_loop(0, CHUNKS_PER_WORKER, chunk_body, 0)


def kernel(src, time, weekday, emb_loc, minute_embed, hour_embed,
           weekday_embed):
    tw = (jnp.repeat(hour_embed, MINUTE_SIZE, axis=0)[:, None, :]
          + jnp.tile(minute_embed, (HOUR_SIZE, 1))[:, None, :]
          + weekday_embed[None, :, :]) * 8.0
    tw = tw.reshape(MINUTE_SIZE * HOUR_SIZE * WEEKDAY_SIZE, D)
    pe = _pe_const(SEQ_LEN, D).reshape(SEQ_LEN * D)

    src_f = src.reshape(TOTAL).astype(jnp.int32)
    time_f = time.reshape(TOTAL).astype(jnp.int32)
    wk_f = weekday.reshape(TOTAL).astype(jnp.int32)

    mesh = plsc.VectorSubcoreMesh(core_axis_name="c", subcore_axis_name="s")
    run = pl.kernel(
        _sc_kernel,
        mesh=mesh,
        compiler_params=pltpu.CompilerParams(use_tc_tiling_on_sc=False),
        out_type=jax.ShapeDtypeStruct((TOTAL, D), jnp.float32),
        scratch_types=[
            pltpu.VMEM((CHUNK,), jnp.int32),
            pltpu.VMEM((CHUNK,), jnp.int32),
            pltpu.VMEM((CHUNK,), jnp.int32),
            pltpu.VMEM((CHUNK,), jnp.int32),
            pltpu.VMEM((CHUNK, D), jnp.float32),
            pltpu.VMEM((CHUNK, D), jnp.float32),
            pltpu.VMEM((D,), jnp.float32),
            pltpu.SemaphoreType.DMA,
            pltpu.SemaphoreType.DMA,
        ],
    )
    out = run(src_f, time_f, wk_f, emb_loc, tw, pe)
    return out.reshape(SEQ_LEN, BATCH, D)


# speedup vs baseline: 7.7286x; 1.3834x over previous
import math

import jax
import jax.numpy as jnp
from jax import lax
from jax.experimental import pallas as pl
from jax.experimental.pallas import tpu as pltpu
from jax.experimental.pallas import tpu_sc as plsc

SEQ_LEN = 200
BATCH = 1024
D = 64
MINUTE_SIZE = 4
HOUR_SIZE = 24
WEEKDAY_SIZE = 7
TW_ROWS = MINUTE_SIZE * HOUR_SIZE * WEEKDAY_SIZE

NUM_CORES = 2
NUM_SUBCORES = 16
NUM_WORKERS = NUM_CORES * NUM_SUBCORES

TOTAL = SEQ_LEN * BATCH
CHUNK = 128
EPW = TOTAL // NUM_WORKERS
CPW = EPW // CHUNK
CHUNKS_PER_ROW = BATCH // CHUNK


def _pe_const(seq_len, emb_size):
    den = jnp.exp(-jnp.arange(0, emb_size, 2).astype(jnp.float32)
                  * math.log(10000.0) / emb_size)
    pos = jnp.arange(0, seq_len).reshape(seq_len, 1).astype(jnp.float32)
    pe = jnp.zeros((seq_len, emb_size), dtype=jnp.float32)
    pe = pe.at[:, 0::2].set(jnp.sin(pos * den))
    pe = pe.at[:, 1::2].set(jnp.cos(pos * den))
    return pe


def _sc_kernel(src_hbm, time_hbm, wk_hbm, emb_hbm, tw_hbm, pe_hbm, out_hbm,
               src_all, twoff_all, tmp_all, pe_all, tw_tile,
               loc0, loc1, ob0, ob1,
               sem_l0, sem_l1, sem_w0, sem_w1):
    wid = lax.axis_index("s") * NUM_CORES + lax.axis_index("c")
    wbase = wid * EPW

    pltpu.sync_copy(src_hbm.at[pl.ds(wbase, EPW)], src_all)
    pltpu.sync_copy(time_hbm.at[pl.ds(wbase, EPW)], twoff_all)
    pltpu.sync_copy(wk_hbm.at[pl.ds(wbase, EPW)], tmp_all)
    pltpu.sync_copy(pe_hbm, pe_all)
    pltpu.sync_copy(tw_hbm, tw_tile)

    def idx_body(i, _):
        sl = pl.ds(i * 16, 16)
        twoff_all[sl] = (twoff_all[sl] * (WEEKDAY_SIZE * D)
                         + tmp_all[sl] * D)
        return 0

    lax.fori_loop(0, EPW // 16, idx_body, 0, unroll=8)

    locs = (loc0, loc1)
    obs = (ob0, ob1)
    sem_l = (sem_l0, sem_l1)
    sem_w = (sem_w0, sem_w1)
    dummy = out_hbm.at[pl.ds(0, CHUNK)]

    def issue_gather(c, b):
        idx = src_all.at[pl.ds(c * CHUNK, CHUNK)]
        pltpu.async_copy(emb_hbm.at[idx], locs[b], sem_l[b])

    def wait_gather(b):
        pltpu.make_async_copy(dummy, locs[b], sem_l[b]).wait()

    def issue_wb(q, b):
        pltpu.async_copy(obs[b], out_hbm.at[pl.ds(q * CHUNK, CHUNK)],
                         sem_w[b])

    def wait_wb(b):
        pltpu.make_async_copy(obs[b], dummy, sem_w[b]).wait()

    def compute_chunk(c, s, b):
        loc = locs[b]
        ob = obs[b]
        pe_regs = [pe_all[pl.ds(s * D + j * 16, 16)] for j in range(D // 16)]
        coff = c * CHUNK

        def grp_body(g, _):
            tvec = twoff_all[pl.ds(coff + g * 16, 16)]
            for k in range(16):
                r = g * 16 + k
                t = tvec[k]
                for j in range(D // 16):
                    sl = pl.ds(j * 16, 16)
                    ob[r, sl] = (loc[r, sl] * 8.0
                                 + tw_tile[pl.ds(t + j * 16, 16)]
                                 + pe_regs[j])
            return 0

        lax.fori_loop(0, CHUNK // 16, grp_body, 0)

    issue_gather(0, 0)
    issue_gather(1, 1)

    def outer(i, _):
        for b in range(2):
            c = i * 2 + b
            q = wid * CPW + c
            s = q // CHUNKS_PER_ROW
            wait_gather(b)

            @pl.when(c >= 2)
            def _():
                wait_wb(b)

            compute_chunk(c, s, b)

            @pl.when(c + 2 < CPW)
            def _():
                issue_gather(c + 2, b)

            issue_wb(q, b)
        return 0

    lax.fori_loop(0, CPW // 2, outer, 0)
    wait_wb(0)
    wait_wb(1)


def kernel(src, time, weekday, emb_loc, minute_embed, hour_embed,
           weekday_embed):
    tw = (jnp.repeat(hour_embed, MINUTE_SIZE, axis=0)[:, None, :]
          + jnp.tile(minute_embed, (HOUR_SIZE, 1))[:, None, :]
          + weekday_embed[None, :, :]) * 8.0
    tw = tw.reshape(TW_ROWS * D)
    pe = _pe_const(SEQ_LEN, D).reshape(SEQ_LEN * D)

    src_f = src.reshape(TOTAL).astype(jnp.int32)
    time_f = time.reshape(TOTAL).astype(jnp.int32)
    wk_f = weekday.reshape(TOTAL).astype(jnp.int32)

    mesh = plsc.VectorSubcoreMesh(core_axis_name="c", subcore_axis_name="s")
    run = pl.kernel(
        _sc_kernel,
        mesh=mesh,
        compiler_params=pltpu.CompilerParams(use_tc_tiling_on_sc=False),
        out_type=jax.ShapeDtypeStruct((TOTAL, D), jnp.float32),
        scratch_types=[
            pltpu.VMEM((EPW,), jnp.int32),
            pltpu.VMEM((EPW,), jnp.int32),
            pltpu.VMEM((EPW,), jnp.int32),
            pltpu.VMEM((SEQ_LEN * D,), jnp.float32),
            pltpu.VMEM((TW_ROWS * D,), jnp.float32),
            pltpu.VMEM((CHUNK, D), jnp.float32),
            pltpu.VMEM((CHUNK, D), jnp.float32),
            pltpu.VMEM((CHUNK, D), jnp.float32),
            pltpu.VMEM((CHUNK, D), jnp.float32),
            pltpu.SemaphoreType.DMA,
            pltpu.SemaphoreType.DMA,
            pltpu.SemaphoreType.DMA,
            pltpu.SemaphoreType.DMA,
        ],
    )
    out = run(src_f, time_f, wk_f, emb_loc, tw, pe)
    return out.reshape(SEQ_LEN, BATCH, D)


# speedup vs baseline: 11.2236x; 1.4522x over previous
import math

import jax
import jax.numpy as jnp
from jax import lax
from jax.experimental import pallas as pl
from jax.experimental.pallas import tpu as pltpu
from jax.experimental.pallas import tpu_sc as plsc

SEQ_LEN = 200
BATCH = 1024
D = 64
MINUTE_SIZE = 4
HOUR_SIZE = 24
WEEKDAY_SIZE = 7
TW_ROWS = MINUTE_SIZE * HOUR_SIZE * WEEKDAY_SIZE

NUM_CORES = 2
NUM_SUBCORES = 16
NUM_WORKERS = NUM_CORES * NUM_SUBCORES

TOTAL = SEQ_LEN * BATCH
CHUNK = 128
EPW = TOTAL // NUM_WORKERS
CPW = EPW // CHUNK
CHUNKS_PER_ROW = BATCH // CHUNK


def _pe_const(seq_len, emb_size):
    den = jnp.exp(-jnp.arange(0, emb_size, 2).astype(jnp.float32)
                  * math.log(10000.0) / emb_size)
    pos = jnp.arange(0, seq_len).reshape(seq_len, 1).astype(jnp.float32)
    pe = jnp.zeros((seq_len, emb_size), dtype=jnp.float32)
    pe = pe.at[:, 0::2].set(jnp.sin(pos * den))
    pe = pe.at[:, 1::2].set(jnp.cos(pos * den))
    return pe


def _sc_kernel(src_hbm, time_hbm, wk_hbm, emb_hbm, tw_hbm, pe_hbm, out_hbm,
               src_all, twidx_all, tmp_all, pe_all, tw_sh,
               loc0, loc1, twr0, twr1, ob0, ob1,
               sem_l0, sem_l1, sem_t0, sem_t1, sem_w0, sem_w1):
    wid = lax.axis_index("s") * NUM_CORES + lax.axis_index("c")
    wbase = wid * EPW

    @pl.when(lax.axis_index("s") == 0)
    def _():
        pltpu.sync_copy(tw_hbm, tw_sh)

    pltpu.sync_copy(src_hbm.at[pl.ds(wbase, EPW)], src_all)
    pltpu.sync_copy(time_hbm.at[pl.ds(wbase, EPW)], twidx_all)
    pltpu.sync_copy(wk_hbm.at[pl.ds(wbase, EPW)], tmp_all)
    pltpu.sync_copy(pe_hbm, pe_all)

    def idx_body(i, _):
        sl = pl.ds(i * 16, 16)
        twidx_all[sl] = twidx_all[sl] * WEEKDAY_SIZE + tmp_all[sl]
        return 0

    lax.fori_loop(0, EPW // 16, idx_body, 0, unroll=8)

    plsc.subcore_barrier()

    locs = (loc0, loc1)
    twrs = (twr0, twr1)
    obs = (ob0, ob1)
    sem_l = (sem_l0, sem_l1)
    sem_t = (sem_t0, sem_t1)
    sem_w = (sem_w0, sem_w1)
    dummy = out_hbm.at[pl.ds(0, CHUNK)]

    def issue_gather(c, b):
        sidx = src_all.at[pl.ds(c * CHUNK, CHUNK)]
        tidx = twidx_all.at[pl.ds(c * CHUNK, CHUNK)]
        pltpu.async_copy(emb_hbm.at[sidx], locs[b], sem_l[b])
        pltpu.async_copy(tw_sh.at[tidx], twrs[b], sem_t[b])

    def wait_gather(b):
        pltpu.make_async_copy(dummy, locs[b], sem_l[b]).wait()
        pltpu.make_async_copy(dummy, twrs[b], sem_t[b]).wait()

    def issue_wb(q, b):
        pltpu.async_copy(obs[b], out_hbm.at[pl.ds(q * CHUNK, CHUNK)],
                         sem_w[b])

    def wait_wb(b):
        pltpu.make_async_copy(obs[b], dummy, sem_w[b]).wait()

    def compute_chunk(s, b):
        loc = locs[b]
        twr = twrs[b]
        ob = obs[b]
        pe_regs = [pe_all[pl.ds(s * D + j * 16, 16)] for j in range(D // 16)]

        def grp_body(g, _):
            for k in range(16):
                r = g * 16 + k
                for j in range(D // 16):
                    sl = pl.ds(j * 16, 16)
                    ob[r, sl] = (loc[r, sl] * 8.0 + twr[r, sl] + pe_regs[j])
            return 0

        lax.fori_loop(0, CHUNK // 16, grp_body, 0)

    issue_gather(0, 0)
    issue_gather(1, 1)

    def outer(i, _):
        for b in range(2):
            c = i * 2 + b
            q = wid * CPW + c
            s = q // CHUNKS_PER_ROW
            wait_gather(b)

            @pl.when(c >= 2)
            def _():
                wait_wb(b)

            compute_chunk(s, b)

            @pl.when(c + 2 < CPW)
            def _():
                issue_gather(c + 2, b)

            issue_wb(q, b)
        return 0

    lax.fori_loop(0, CPW // 2, outer, 0)
    wait_wb(0)
    wait_wb(1)


def kernel(src, time, weekday, emb_loc, minute_embed, hour_embed,
           weekday_embed):
    tw = (jnp.repeat(hour_embed, MINUTE_SIZE, axis=0)[:, None, :]
          + jnp.tile(minute_embed, (HOUR_SIZE, 1))[:, None, :]
          + weekday_embed[None, :, :]) * 8.0
    tw = tw.reshape(TW_ROWS, D)
    pe = _pe_const(SEQ_LEN, D).reshape(SEQ_LEN * D)

    src_f = src.reshape(TOTAL).astype(jnp.int32)
    time_f = time.reshape(TOTAL).astype(jnp.int32)
    wk_f = weekday.reshape(TOTAL).astype(jnp.int32)

    mesh = plsc.VectorSubcoreMesh(core_axis_name="c", subcore_axis_name="s")
    run = pl.kernel(
        _sc_kernel,
        mesh=mesh,
        compiler_params=pltpu.CompilerParams(use_tc_tiling_on_sc=False),
        out_type=jax.ShapeDtypeStruct((TOTAL, D), jnp.float32),
        scratch_types=[
            pltpu.VMEM((EPW,), jnp.int32),
            pltpu.VMEM((EPW,), jnp.int32),
            pltpu.VMEM((EPW,), jnp.int32),
            pltpu.VMEM((SEQ_LEN * D,), jnp.float32),
            pltpu.VMEM_SHARED((TW_ROWS, D), jnp.float32),
            pltpu.VMEM((CHUNK, D), jnp.float32),
            pltpu.VMEM((CHUNK, D), jnp.float32),
            pltpu.VMEM((CHUNK, D), jnp.float32),
            pltpu.VMEM((CHUNK, D), jnp.float32),
            pltpu.VMEM((CHUNK, D), jnp.float32),
            pltpu.VMEM((CHUNK, D), jnp.float32),
            pltpu.SemaphoreType.DMA,
            pltpu.SemaphoreType.DMA,
            pltpu.SemaphoreType.DMA,
            pltpu.SemaphoreType.DMA,
            pltpu.SemaphoreType.DMA,
            pltpu.SemaphoreType.DMA,
        ],
    )
    out = run(src_f, time_f, wk_f, emb_loc, tw, pe)
    return out.reshape(SEQ_LEN, BATCH, D)
